# baseline (speedup 1.0000x reference)
import functools

import jax
import jax.numpy as jnp
from jax import lax
from jax.experimental import pallas as pl
from jax.experimental.pallas import tpu as pltpu
from jax.experimental.pallas import tpu_sc as plsc

N_U = 25000
N_I = 25000
EMB = 64
HEMB = 32
E = 800000

HALF = 25088
NTOT = 2 * HALF
PADG = HALF - N_U
B = 128
NBT = 400
NBALL = 16 * NBT
E_PAD = NBALL * B
KBLK = 8
NHALF = NBT // (2 * KBLK)
STRIPE = NTOT // 16
ZCH = 196
NZ = STRIPE // ZCH


def _scale_rows(rows, pk, jb):
    def group(g, carry):
        wv = lax.bitcast_convert_type(pk[jb, 2, pl.ds(g * 16, 16)],
                                      jnp.float32)
        for j in range(16):
            ws = wv.at[jnp.full((16,), j, jnp.int32)].get(
                mode="promise_in_bounds")
            e = g * 16 + j
            for q in range(HEMB // 16):
                sl = pl.ds(q * 16, 16)
                rows[e, sl] = rows[e, sl] * ws
        return carry

    lax.fori_loop(0, B // 16, group, 0)


def _one_layer(final, packed_hbm, tin_hbm, sin_hbm, tout_hbm, sout_hbm,
               acc, pka, pkb, rows0, rows1, zbuf, sbuf, sem0, sem1,
               sempa, sempb):
    c = lax.axis_index("c")
    s = lax.axis_index("s")

    z = jnp.zeros((16,), jnp.float32)

    def zrow(r, carry):
        for q in range(HEMB // 16):
            zbuf[r, pl.ds(q * 16, 16)] = z
        return carry

    lax.fori_loop(0, ZCH, zrow, 0)
    for k in range(NZ):
        pltpu.sync_copy(zbuf, acc.at[pl.ds(s * STRIPE + k * ZCH, ZCH)])
    plsc.subcore_barrier()

    base = c * NBALL + s * NBT

    pltpu.sync_copy(packed_hbm.at[pl.ds(base, KBLK)], pka)
    pltpu.async_copy(packed_hbm.at[pl.ds(base + KBLK, KBLK)], pkb, sempa)
    pltpu.async_copy(tin_hbm.at[pka.at[0, 0]], rows0, sem0)

    def pair(i, carry):
        blk = base + 2 * KBLK * i

        def run_block(pk, pknext, nextfirst, semnext, guard_next):
            for j in range(KBLK):
                rj, sj = (rows0, sem0) if j % 2 == 0 else (rows1, sem1)
                rn, sn = (rows1, sem1) if j % 2 == 0 else (rows0, sem0)
                if j < KBLK - 1:
                    pltpu.async_copy(tin_hbm.at[pk.at[j + 1, 0]], rn, sn)
                else:
                    def arm_next():
                        pltpu.make_async_copy(
                            packed_hbm.at[pl.ds(nextfirst, KBLK)],
                            pknext, semnext).wait()
                        pltpu.async_copy(tin_hbm.at[pknext.at[0, 0]], rn, sn)

                    if guard_next is None:
                        arm_next()
                    else:
                        pl.when(guard_next)(arm_next)
                pltpu.make_async_copy(tin_hbm.at[pk.at[j, 0]], rj, sj).wait()
                _scale_rows(rj, pk, j)
                pltpu.sync_copy(rj, acc.at[pk.at[j, 1]], add=True)

        run_block(pka, pkb, blk + KBLK, sempa, None)

        @pl.when(i < NHALF - 1)
        def _():
            pltpu.async_copy(packed_hbm.at[pl.ds(blk + 2 * KBLK, KBLK)],
                             pka, sempb)

        run_block(pkb, pka, blk + 2 * KBLK, sempb, i < NHALF - 1)

        @pl.when(i < NHALF - 1)
        def _():
            pltpu.async_copy(packed_hbm.at[pl.ds(blk + 3 * KBLK, KBLK)],
                             pkb, sempa)
        return carry

    lax.fori_loop(0, NHALF, pair, 0)
    plsc.subcore_barrier()

    row0 = c * NTOT + s * STRIPE
    if not final:
        pltpu.sync_copy(acc.at[pl.ds(s * STRIPE, STRIPE)],
                        tout_hbm.at[pl.ds(row0, STRIPE)])
    for k in range(NZ):
        pltpu.sync_copy(acc.at[pl.ds(s * STRIPE + k * ZCH, ZCH)], zbuf)
        pltpu.sync_copy(sin_hbm.at[pl.ds(row0 + k * ZCH, ZCH)], sbuf)

        def srow(r, carry):
            for q in range(HEMB // 16):
                sl = pl.ds(q * 16, 16)
                v = sbuf[r, sl] + zbuf[r, sl]
                if final:
                    v = v * 0.25
                sbuf[r, sl] = v
            return carry

        lax.fori_loop(0, ZCH, srow, 0)
        pltpu.sync_copy(sbuf, sout_hbm.at[pl.ds(row0 + k * ZCH, ZCH)])
    plsc.subcore_barrier()


def _fused_body(packed_hbm, t0_hbm, tmpa_hbm, tmpb_hbm, sum_hbm,
                acc, pka, pkb, rows0, rows1, zbuf, sbuf, sem0, sem1,
                sempa, sempb):
    bufs = (acc, pka, pkb, rows0, rows1, zbuf, sbuf, sem0, sem1,
            sempa, sempb)
    _one_layer(False, packed_hbm, t0_hbm, t0_hbm, tmpa_hbm, sum_hbm, *bufs)
    _one_layer(False, packed_hbm, tmpa_hbm, sum_hbm, tmpb_hbm, sum_hbm, *bufs)
    _one_layer(True, packed_hbm, tmpb_hbm, sum_hbm, None, sum_hbm, *bufs)


def _make_fused():
    mesh = plsc.VectorSubcoreMesh(core_axis_name="c", subcore_axis_name="s")
    return pl.kernel(
        _fused_body,
        out_type=(
            jax.ShapeDtypeStruct((2 * NTOT, HEMB), jnp.float32),
            jax.ShapeDtypeStruct((2 * NTOT, HEMB), jnp.float32),
            jax.ShapeDtypeStruct((2 * NTOT, HEMB), jnp.float32),
        ),
        mesh=mesh,
        compiler_params=pltpu.CompilerParams(use_tc_tiling_on_sc=False),
        scratch_types=[
            pltpu.VMEM_SHARED((NTOT, HEMB), jnp.float32),
            pltpu.VMEM((KBLK, 3, B), jnp.int32),
            pltpu.VMEM((KBLK, 3, B), jnp.int32),
            pltpu.VMEM((B, HEMB), jnp.float32),
            pltpu.VMEM((B, HEMB), jnp.float32),
            pltpu.VMEM((ZCH, HEMB), jnp.float32),
            pltpu.VMEM((ZCH, HEMB), jnp.float32),
            pltpu.SemaphoreType.DMA,
            pltpu.SemaphoreType.DMA,
            pltpu.SemaphoreType.DMA,
            pltpu.SemaphoreType.DMA,
        ],
        name="lightgcn_fused3",
    )


def kernel(edge_index, edge_weight, user_emb, item_emb):
    src = edge_index[1].astype(jnp.int32)
    dst = edge_index[0].astype(jnp.int32)
    src_p = src + PADG * (src >= N_U).astype(jnp.int32)
    dst_p = dst + PADG * (dst >= N_U).astype(jnp.int32)
    wbits = lax.bitcast_convert_type(edge_weight.astype(jnp.float32), jnp.int32)

    def pack(src_c):
        p = jnp.stack([src_c, dst_p, wbits])
        p = jnp.pad(p, ((0, 0), (0, E_PAD - E)))
        return p.reshape(3, NBALL, B).transpose(1, 0, 2)

    packed = jnp.concatenate([pack(src_p), pack(src_p + NTOT)], axis=0)

    zpad = jnp.zeros((PADG, EMB), jnp.float32)
    emb0 = jnp.concatenate([user_emb, zpad, item_emb, zpad], axis=0)
    split0 = jnp.concatenate([emb0[:, :HEMB], emb0[:, HEMB:]], axis=0)

    _, _, sum3 = _make_fused()(packed, split0)

    out = jnp.concatenate([sum3[:NTOT], sum3[NTOT:]], axis=1)
    return (out[:N_U], out[HALF:HALF + N_I])


# speedup vs baseline: 1.3427x; 1.3427x over previous
import jax
import jax.numpy as jnp
from jax import lax
from jax.experimental import pallas as pl
from jax.experimental.pallas import tpu as pltpu
from jax.experimental.pallas import tpu_sc as plsc

N_U = 25000
N_I = 25000
EMB = 64
HEMB = 32
E = 800000

HALF = 25088
NTOT = 2 * HALF
PADG = HALF - N_U
B = 128
NBT = 392
NBALL = 16 * NBT
E_PAD = NBALL * B
STRIPE = NTOT // 16
ZCH = 98
NZ = STRIPE // ZCH


def _scale_rows(rows, pk):
    def group(g, carry):
        wv = lax.bitcast_convert_type(pk[2, pl.ds(g * 16, 16)], jnp.float32)
        for j in range(16):
            ws = wv.at[jnp.full((16,), j, jnp.int32)].get(
                mode="promise_in_bounds")
            e = g * 16 + j
            for q in range(HEMB // 16):
                sl = pl.ds(q * 16, 16)
                rows[e, sl] = rows[e, sl] * ws
        return carry

    lax.fori_loop(0, B // 16, group, 0)


def _one_layer(final, packed_hbm, tin_hbm, sin_hbm, tout_hbm, sout_hbm,
               acc, pk0, pk1, rows0, rows1, zerobuf, sibuf0, sibuf1,
               sobuf0, sobuf1, sem0, sem1, sempa, sempb, semw0, semw1,
               semtab):
    c = lax.axis_index("c")
    s = lax.axis_index("s")

    for k in range(NZ):
        pltpu.sync_copy(zerobuf, acc.at[pl.ds(s * STRIPE + k * ZCH, ZCH)])
    plsc.subcore_barrier()

    base = c * NBALL + s * NBT

    pltpu.sync_copy(packed_hbm.at[base], pk0)
    pltpu.async_copy(tin_hbm.at[pk0.at[0]], rows0, sem0)

    def pair(i, carry):
        b = 2 * i
        pltpu.sync_copy(packed_hbm.at[base + b + 1], pk1)
        pltpu.async_copy(tin_hbm.at[pk1.at[0]], rows1, sem1)
        pltpu.make_async_copy(tin_hbm.at[pk0.at[0]], rows0, sem0).wait()
        _scale_rows(rows0, pk0)
        pltpu.sync_copy(rows0, acc.at[pk0.at[1]], add=True)

        @pl.when(i < NBT // 2 - 1)
        def _():
            pltpu.sync_copy(packed_hbm.at[base + b + 2], pk0)
            pltpu.async_copy(tin_hbm.at[pk0.at[0]], rows0, sem0)

        pltpu.make_async_copy(tin_hbm.at[pk1.at[0]], rows1, sem1).wait()
        _scale_rows(rows1, pk1)
        pltpu.sync_copy(rows1, acc.at[pk1.at[1]], add=True)
        return carry

    lax.fori_loop(0, NBT // 2, pair, 0)
    plsc.subcore_barrier()

    row0 = c * NTOT + s * STRIPE
    if not final:
        pltpu.async_copy(acc.at[pl.ds(s * STRIPE, STRIPE)],
                         tout_hbm.at[pl.ds(row0, STRIPE)], semtab)

    pltpu.async_copy(sin_hbm.at[pl.ds(row0, ZCH)], sibuf0, sempa)
    for k in range(NZ):
        si, ssem = (sibuf0, sempa) if k % 2 == 0 else (sibuf1, sempb)
        so, wsem = (sobuf0, semw0) if k % 2 == 0 else (sobuf1, semw1)
        ni, nsem = (sibuf1, sempb) if k % 2 == 0 else (sibuf0, sempa)
        if k + 1 < NZ:
            pltpu.async_copy(sin_hbm.at[pl.ds(row0 + (k + 1) * ZCH, ZCH)],
                             ni, nsem)
        if k >= 2:
            pltpu.make_async_copy(
                so, sout_hbm.at[pl.ds(row0 + (k - 2) * ZCH, ZCH)],
                wsem).wait()
        pltpu.sync_copy(acc.at[pl.ds(s * STRIPE + k * ZCH, ZCH)], so)
        pltpu.make_async_copy(sin_hbm.at[pl.ds(row0 + k * ZCH, ZCH)],
                              si, ssem).wait()

        def srow(r, carry):
            for q in range(HEMB // 16):
                sl = pl.ds(q * 16, 16)
                v = so[r, sl] + si[r, sl]
                if final:
                    v = v * 0.25
                so[r, sl] = v
            return carry

        lax.fori_loop(0, ZCH, srow, 0)
        pltpu.async_copy(so, sout_hbm.at[pl.ds(row0 + k * ZCH, ZCH)], wsem)

    pltpu.make_async_copy(
        sobuf0, sout_hbm.at[pl.ds(row0 + (NZ - 2) * ZCH, ZCH)], semw0).wait()
    pltpu.make_async_copy(
        sobuf1, sout_hbm.at[pl.ds(row0 + (NZ - 1) * ZCH, ZCH)], semw1).wait()
    if not final:
        pltpu.make_async_copy(acc.at[pl.ds(s * STRIPE, STRIPE)],
                              tout_hbm.at[pl.ds(row0, STRIPE)], semtab).wait()
    plsc.subcore_barrier()


def _fused_body(packed_hbm, t0_hbm, tmpa_hbm, tmpb_hbm, sum_hbm,
                acc, pk0, pk1, rows0, rows1, zerobuf, sibuf0, sibuf1,
                sobuf0, sobuf1, sem0, sem1, sempa, sempb, semw0, semw1,
                semtab):
    bufs = (acc, pk0, pk1, rows0, rows1, zerobuf, sibuf0, sibuf1,
            sobuf0, sobuf1, sem0, sem1, sempa, sempb, semw0, semw1,
            semtab)

    z = jnp.zeros((16,), jnp.float32)

    def zrow(r, carry):
        for q in range(HEMB // 16):
            zerobuf[r, pl.ds(q * 16, 16)] = z
        return carry

    lax.fori_loop(0, ZCH, zrow, 0)

    _one_layer(False, packed_hbm, t0_hbm, t0_hbm, tmpa_hbm, sum_hbm, *bufs)
    _one_layer(False, packed_hbm, tmpa_hbm, sum_hbm, tmpb_hbm, sum_hbm, *bufs)
    _one_layer(True, packed_hbm, tmpb_hbm, sum_hbm, None, sum_hbm, *bufs)


def _make_fused():
    mesh = plsc.VectorSubcoreMesh(core_axis_name="c", subcore_axis_name="s")
    return pl.kernel(
        _fused_body,
        out_type=(
            jax.ShapeDtypeStruct((2 * NTOT, HEMB), jnp.float32),
            jax.ShapeDtypeStruct((2 * NTOT, HEMB), jnp.float32),
            jax.ShapeDtypeStruct((2 * NTOT, HEMB), jnp.float32),
        ),
        mesh=mesh,
        compiler_params=pltpu.CompilerParams(use_tc_tiling_on_sc=False),
        scratch_types=[
            pltpu.VMEM_SHARED((NTOT, HEMB), jnp.float32),
            pltpu.VMEM((3, B), jnp.int32),
            pltpu.VMEM((3, B), jnp.int32),
            pltpu.VMEM((B, HEMB), jnp.float32),
            pltpu.VMEM((B, HEMB), jnp.float32),
            pltpu.VMEM((ZCH, HEMB), jnp.float32),
            pltpu.VMEM((ZCH, HEMB), jnp.float32),
            pltpu.VMEM((ZCH, HEMB), jnp.float32),
            pltpu.VMEM((ZCH, HEMB), jnp.float32),
            pltpu.VMEM((ZCH, HEMB), jnp.float32),
            pltpu.SemaphoreType.DMA,
            pltpu.SemaphoreType.DMA,
            pltpu.SemaphoreType.DMA,
            pltpu.SemaphoreType.DMA,
            pltpu.SemaphoreType.DMA,
            pltpu.SemaphoreType.DMA,
            pltpu.SemaphoreType.DMA,
        ],
        name="lightgcn_fused3",
    )


def kernel(edge_index, edge_weight, user_emb, item_emb):
    src = edge_index[1].astype(jnp.int32)
    dst = edge_index[0].astype(jnp.int32)
    src_p = src + PADG * (src >= N_U).astype(jnp.int32)
    dst_p = dst + PADG * (dst >= N_U).astype(jnp.int32)
    wbits = lax.bitcast_convert_type(edge_weight.astype(jnp.float32), jnp.int32)

    def pack(src_c):
        p = jnp.stack([src_c, dst_p, wbits])
        p = jnp.pad(p, ((0, 0), (0, E_PAD - E)))
        return p.reshape(3, NBALL, B).transpose(1, 0, 2)

    packed = jnp.concatenate([pack(src_p), pack(src_p + NTOT)], axis=0)

    zpad = jnp.zeros((PADG, EMB), jnp.float32)
    emb0 = jnp.concatenate([user_emb, zpad, item_emb, zpad], axis=0)
    split0 = jnp.concatenate([emb0[:, :HEMB], emb0[:, HEMB:]], axis=0)

    _, _, sum3 = _make_fused()(packed, split0)

    out = jnp.concatenate([sum3[:NTOT], sum3[NTOT:]], axis=1)
    return (out[:N_U], out[HALF:HALF + N_I])


# speedup vs baseline: 1.4026x; 1.0446x over previous
import jax
import jax.numpy as jnp
from jax import lax
from jax.experimental import pallas as pl
from jax.experimental.pallas import tpu as pltpu
from jax.experimental.pallas import tpu_sc as plsc

N_U = 25000
N_I = 25000
EMB = 64
HEMB = 32
E = 800000

HALF = 25088
NTOT = 2 * HALF
PADG = HALF - N_U
B = 128
NBT = 392
NBALL = 16 * NBT
E_PAD = NBALL * B
STRIPE = NTOT // 16
ZCH = 98
NZ = STRIPE // ZCH


def _scale_rows(rows, pk):
    def group(g, carry):
        wv = lax.bitcast_convert_type(pk[2, pl.ds(g * 16, 16)], jnp.float32)
        for j in range(16):
            ws = wv.at[jnp.full((16,), j, jnp.int32)].get(
                mode="promise_in_bounds")
            e = g * 16 + j
            for q in range(HEMB // 16):
                sl = pl.ds(q * 16, 16)
                rows[e, sl] = rows[e, sl] * ws
        return carry

    lax.fori_loop(0, B // 16, group, 0)


def _one_layer(final, packed_hbm, tin_hbm, sin_hbm, tout_hbm, sout_hbm,
               acc, pk0, pk1, rows0, rows1, zerobuf, sibuf0, sibuf1,
               sobuf0, sobuf1, sem0, sem1, sempa, sempb, semw0, semw1,
               semtab):
    c = lax.axis_index("c")
    s = lax.axis_index("s")

    for k in range(NZ):
        pltpu.sync_copy(zerobuf, acc.at[pl.ds(s * STRIPE + k * ZCH, ZCH)])
    plsc.subcore_barrier()

    base = c * NBALL + s * NBT

    pltpu.sync_copy(packed_hbm.at[base], pk0)
    pltpu.async_copy(tin_hbm.at[pk0.at[0]], rows0, sem0)

    def pair(i, carry):
        b = 2 * i
        pltpu.sync_copy(packed_hbm.at[base + b + 1], pk1)
        pltpu.async_copy(tin_hbm.at[pk1.at[0]], rows1, sem1)
        pltpu.make_async_copy(tin_hbm.at[pk0.at[0]], rows0, sem0).wait()
        _scale_rows(rows0, pk0)
        pltpu.sync_copy(rows0, acc.at[pk0.at[1]], add=True)

        @pl.when(i < NBT // 2 - 1)
        def _():
            pltpu.sync_copy(packed_hbm.at[base + b + 2], pk0)
            pltpu.async_copy(tin_hbm.at[pk0.at[0]], rows0, sem0)

        pltpu.make_async_copy(tin_hbm.at[pk1.at[0]], rows1, sem1).wait()
        _scale_rows(rows1, pk1)
        pltpu.sync_copy(rows1, acc.at[pk1.at[1]], add=True)
        return carry

    lax.fori_loop(0, NBT // 2, pair, 0)
    plsc.subcore_barrier()

    row0 = c * NTOT + s * STRIPE
    if not final:
        pltpu.async_copy(acc.at[pl.ds(s * STRIPE, STRIPE)],
                         tout_hbm.at[pl.ds(row0, STRIPE)], semtab)

    pltpu.async_copy(sin_hbm.at[pl.ds(row0, ZCH)], sibuf0, sempa)
    for k in range(NZ):
        si, ssem = (sibuf0, sempa) if k % 2 == 0 else (sibuf1, sempb)
        so, wsem = (sobuf0, semw0) if k % 2 == 0 else (sobuf1, semw1)
        ni, nsem = (sibuf1, sempb) if k % 2 == 0 else (sibuf0, sempa)
        if k + 1 < NZ:
            pltpu.async_copy(sin_hbm.at[pl.ds(row0 + (k + 1) * ZCH, ZCH)],
                             ni, nsem)
        if k >= 2:
            pltpu.make_async_copy(
                so, sout_hbm.at[pl.ds(row0 + (k - 2) * ZCH, ZCH)],
                wsem).wait()
        pltpu.sync_copy(acc.at[pl.ds(s * STRIPE + k * ZCH, ZCH)], so)
        pltpu.make_async_copy(sin_hbm.at[pl.ds(row0 + k * ZCH, ZCH)],
                              si, ssem).wait()

        def srow(r, carry):
            for q in range(HEMB // 16):
                sl = pl.ds(q * 16, 16)
                v = so[r, sl] + si[r, sl]
                if final:
                    v = v * 0.25
                so[r, sl] = v
            return carry

        lax.fori_loop(0, ZCH, srow, 0)
        pltpu.async_copy(so, sout_hbm.at[pl.ds(row0 + k * ZCH, ZCH)], wsem)

    pltpu.make_async_copy(
        sobuf0, sout_hbm.at[pl.ds(row0 + (NZ - 2) * ZCH, ZCH)], semw0).wait()
    pltpu.make_async_copy(
        sobuf1, sout_hbm.at[pl.ds(row0 + (NZ - 1) * ZCH, ZCH)], semw1).wait()
    if not final:
        pltpu.make_async_copy(acc.at[pl.ds(s * STRIPE, STRIPE)],
                              tout_hbm.at[pl.ds(row0, STRIPE)], semtab).wait()
    plsc.subcore_barrier()


def _fused_body(packed_hbm, t0_hbm, tmpa_hbm, tmpb_hbm, sum_hbm,
                acc, pk0, pk1, rows0, rows1, zerobuf, sibuf0, sibuf1,
                sobuf0, sobuf1, sem0, sem1, sempa, sempb, semw0, semw1,
                semtab):
    bufs = (acc, pk0, pk1, rows0, rows1, zerobuf, sibuf0, sibuf1,
            sobuf0, sobuf1, sem0, sem1, sempa, sempb, semw0, semw1,
            semtab)

    z = jnp.zeros((16,), jnp.float32)

    def zrow(r, carry):
        for q in range(HEMB // 16):
            zerobuf[r, pl.ds(q * 16, 16)] = z
        return carry

    lax.fori_loop(0, ZCH, zrow, 0)

    _one_layer(False, packed_hbm, t0_hbm, t0_hbm, tmpa_hbm, sum_hbm, *bufs)
    _one_layer(False, packed_hbm, tmpa_hbm, sum_hbm, tmpb_hbm, sum_hbm, *bufs)
    _one_layer(True, packed_hbm, tmpb_hbm, sum_hbm, None, sum_hbm, *bufs)


def _make_fused():
    mesh = plsc.VectorSubcoreMesh(core_axis_name="c", subcore_axis_name="s")
    return pl.kernel(
        _fused_body,
        out_type=(
            jax.ShapeDtypeStruct((2 * NTOT, HEMB), jnp.float32),
            jax.ShapeDtypeStruct((2 * NTOT, HEMB), jnp.float32),
            jax.ShapeDtypeStruct((2 * NTOT, HEMB), jnp.float32),
        ),
        mesh=mesh,
        compiler_params=pltpu.CompilerParams(use_tc_tiling_on_sc=False),
        scratch_types=[
            pltpu.VMEM_SHARED((NTOT, HEMB), jnp.float32),
            pltpu.VMEM((3, B), jnp.int32),
            pltpu.VMEM((3, B), jnp.int32),
            pltpu.VMEM((B, HEMB), jnp.float32),
            pltpu.VMEM((B, HEMB), jnp.float32),
            pltpu.VMEM((ZCH, HEMB), jnp.float32),
            pltpu.VMEM((ZCH, HEMB), jnp.float32),
            pltpu.VMEM((ZCH, HEMB), jnp.float32),
            pltpu.VMEM((ZCH, HEMB), jnp.float32),
            pltpu.VMEM((ZCH, HEMB), jnp.float32),
            pltpu.SemaphoreType.DMA,
            pltpu.SemaphoreType.DMA,
            pltpu.SemaphoreType.DMA,
            pltpu.SemaphoreType.DMA,
            pltpu.SemaphoreType.DMA,
            pltpu.SemaphoreType.DMA,
            pltpu.SemaphoreType.DMA,
        ],
        name="lightgcn_fused3",
    )


def kernel(edge_index, edge_weight, user_emb, item_emb):
    src = edge_index[1].astype(jnp.int32)
    dst = edge_index[0].astype(jnp.int32)
    src_p = src + PADG * (src >= N_U).astype(jnp.int32)
    dst_p = dst + PADG * (dst >= N_U).astype(jnp.int32)
    wbits = lax.bitcast_convert_type(edge_weight.astype(jnp.float32), jnp.int32)

    pad_idx = N_U + (jnp.arange(E_PAD - E, dtype=jnp.int32) % PADG)

    def pack(src_c):
        p = jnp.stack([src_c, dst_p, wbits])
        pads = jnp.stack([pad_idx, pad_idx,
                          jnp.zeros((E_PAD - E,), jnp.int32)])
        p = jnp.concatenate([p, pads], axis=1)
        return p.reshape(3, NBALL, B).transpose(1, 0, 2)

    packed = jnp.concatenate([pack(src_p), pack(src_p + NTOT)], axis=0)

    zpad = jnp.zeros((PADG, EMB), jnp.float32)
    emb0 = jnp.concatenate([user_emb, zpad, item_emb, zpad], axis=0)
    split0 = jnp.concatenate([emb0[:, :HEMB], emb0[:, HEMB:]], axis=0)

    _, _, sum3 = _make_fused()(packed, split0)

    out = jnp.concatenate([sum3[:NTOT], sum3[NTOT:]], axis=1)
    return (out[:N_U], out[HALF:HALF + N_I])


# speedup vs baseline: 1.5942x; 1.1366x over previous
import jax
import jax.numpy as jnp
from jax import lax
from jax.experimental import pallas as pl
from jax.experimental.pallas import tpu as pltpu
from jax.experimental.pallas import tpu_sc as plsc

N_U = 25000
N_I = 25000
EMB = 64
HEMB = 32
E = 800000

HALF = 25088
NTOT = 2 * HALF
PADG = HALF - N_U
B = 128
NBT = 396
NBALL = 16 * NBT
E_PAD = NBALL * B
STRIPE = NTOT // 16
ZCH = 98
NZ = STRIPE // ZCH


def _scale_rows(rows, pk):
    def group(g, carry):
        wv = lax.bitcast_convert_type(pk[2, pl.ds(g * 16, 16)], jnp.float32)
        for j in range(16):
            ws = wv.at[jnp.full((16,), j, jnp.int32)].get(
                mode="promise_in_bounds")
            e = g * 16 + j
            for q in range(HEMB // 16):
                sl = pl.ds(q * 16, 16)
                rows[e, sl] = rows[e, sl] * ws
        return carry

    lax.fori_loop(0, B // 16, group, 0)


def _one_layer(final, packed_hbm, tin_hbm, sin_hbm, tout_hbm, sout_hbm,
               acc, pk0, pk1, pk2, rows0, rows1, rows2, zerobuf,
               sibuf0, sibuf1, sobuf0, sobuf1, sem0, sem1, sem2,
               sempa, sempb, semw0, semw1, semw2, semtab):
    c = lax.axis_index("c")
    s = lax.axis_index("s")

    for k in range(NZ):
        pltpu.sync_copy(zerobuf, acc.at[pl.ds(s * STRIPE + k * ZCH, ZCH)])
    plsc.subcore_barrier()

    base = c * NBALL + s * NBT
    pks = (pk0, pk1, pk2)
    rows = (rows0, rows1, rows2)
    gsems = (sem0, sem1, sem2)
    wsems = (semw0, semw1, semw2)

    pltpu.sync_copy(packed_hbm.at[base], pk0)
    pltpu.async_copy(tin_hbm.at[pk0.at[0]], rows0, sem0)

    def triple(i, carry):
        for j in range(3):
            b = 3 * i + j
            n = (j + 1) % 3

            @pl.when(b + 1 < NBT)
            def _():
                @pl.when(b >= 2)
                def _():
                    pltpu.make_async_copy(
                        rows[n], acc.at[pks[n].at[1]], wsems[n]).wait()

                pltpu.sync_copy(packed_hbm.at[base + b + 1], pks[n])
                pltpu.async_copy(tin_hbm.at[pks[n].at[0]], rows[n], gsems[n])

            pltpu.make_async_copy(tin_hbm.at[pks[j].at[0]], rows[j],
                                  gsems[j]).wait()
            _scale_rows(rows[j], pks[j])
            pltpu.async_copy(rows[j], acc.at[pks[j].at[1]], wsems[j],
                             add=True)
        return carry

    lax.fori_loop(0, NBT // 3, triple, 0)
    for j in range(3):
        pltpu.make_async_copy(rows[j], acc.at[pks[j].at[1]],
                              wsems[j]).wait()
    plsc.subcore_barrier()

    row0 = c * NTOT + s * STRIPE
    if not final:
        pltpu.async_copy(acc.at[pl.ds(s * STRIPE, STRIPE)],
                         tout_hbm.at[pl.ds(row0, STRIPE)], semtab)

    pltpu.async_copy(sin_hbm.at[pl.ds(row0, ZCH)], sibuf0, sempa)
    for k in range(NZ):
        si, ssem = (sibuf0, sempa) if k % 2 == 0 else (sibuf1, sempb)
        so, wsem = (sobuf0, semw0) if k % 2 == 0 else (sobuf1, semw1)
        ni, nsem = (sibuf1, sempb) if k % 2 == 0 else (sibuf0, sempa)
        if k + 1 < NZ:
            pltpu.async_copy(sin_hbm.at[pl.ds(row0 + (k + 1) * ZCH, ZCH)],
                             ni, nsem)
        if k >= 2:
            pltpu.make_async_copy(
                so, sout_hbm.at[pl.ds(row0 + (k - 2) * ZCH, ZCH)],
                wsem).wait()
        pltpu.sync_copy(acc.at[pl.ds(s * STRIPE + k * ZCH, ZCH)], so)
        pltpu.make_async_copy(sin_hbm.at[pl.ds(row0 + k * ZCH, ZCH)],
                              si, ssem).wait()

        def srow(r, carry):
            for q in range(HEMB // 16):
                sl = pl.ds(q * 16, 16)
                v = so[r, sl] + si[r, sl]
                if final:
                    v = v * 0.25
                so[r, sl] = v
            return carry

        lax.fori_loop(0, ZCH, srow, 0)
        pltpu.async_copy(so, sout_hbm.at[pl.ds(row0 + k * ZCH, ZCH)], wsem)

    pltpu.make_async_copy(
        sobuf0, sout_hbm.at[pl.ds(row0 + (NZ - 2) * ZCH, ZCH)], semw0).wait()
    pltpu.make_async_copy(
        sobuf1, sout_hbm.at[pl.ds(row0 + (NZ - 1) * ZCH, ZCH)], semw1).wait()
    if not final:
        pltpu.make_async_copy(acc.at[pl.ds(s * STRIPE, STRIPE)],
                              tout_hbm.at[pl.ds(row0, STRIPE)], semtab).wait()
    plsc.subcore_barrier()


def _fused_body(packed_hbm, t0_hbm, tmpa_hbm, tmpb_hbm, sum_hbm,
                acc, pk0, pk1, pk2, rows0, rows1, rows2, zerobuf,
                sibuf0, sibuf1, sobuf0, sobuf1, sem0, sem1, sem2,
                sempa, sempb, semw0, semw1, semw2, semtab):
    bufs = (acc, pk0, pk1, pk2, rows0, rows1, rows2, zerobuf,
            sibuf0, sibuf1, sobuf0, sobuf1, sem0, sem1, sem2,
            sempa, sempb, semw0, semw1, semw2, semtab)

    z = jnp.zeros((16,), jnp.float32)

    def zrow(r, carry):
        for q in range(HEMB // 16):
            zerobuf[r, pl.ds(q * 16, 16)] = z
        return carry

    lax.fori_loop(0, ZCH, zrow, 0)

    _one_layer(False, packed_hbm, t0_hbm, t0_hbm, tmpa_hbm, sum_hbm, *bufs)
    _one_layer(False, packed_hbm, tmpa_hbm, sum_hbm, tmpb_hbm, sum_hbm, *bufs)
    _one_layer(True, packed_hbm, tmpb_hbm, sum_hbm, None, sum_hbm, *bufs)


def _make_fused():
    mesh = plsc.VectorSubcoreMesh(core_axis_name="c", subcore_axis_name="s")
    return pl.kernel(
        _fused_body,
        out_type=(
            jax.ShapeDtypeStruct((2 * NTOT, HEMB), jnp.float32),
            jax.ShapeDtypeStruct((2 * NTOT, HEMB), jnp.float32),
            jax.ShapeDtypeStruct((2 * NTOT, HEMB), jnp.float32),
        ),
        mesh=mesh,
        compiler_params=pltpu.CompilerParams(use_tc_tiling_on_sc=False),
        scratch_types=[
            pltpu.VMEM_SHARED((NTOT, HEMB), jnp.float32),
            pltpu.VMEM((3, B), jnp.int32),
            pltpu.VMEM((3, B), jnp.int32),
            pltpu.VMEM((3, B), jnp.int32),
            pltpu.VMEM((B, HEMB), jnp.float32),
            pltpu.VMEM((B, HEMB), jnp.float32),
            pltpu.VMEM((B, HEMB), jnp.float32),
            pltpu.VMEM((ZCH, HEMB), jnp.float32),
            pltpu.VMEM((ZCH, HEMB), jnp.float32),
            pltpu.VMEM((ZCH, HEMB), jnp.float32),
            pltpu.VMEM((ZCH, HEMB), jnp.float32),
            pltpu.VMEM((ZCH, HEMB), jnp.float32),
            pltpu.SemaphoreType.DMA,
            pltpu.SemaphoreType.DMA,
            pltpu.SemaphoreType.DMA,
            pltpu.SemaphoreType.DMA,
            pltpu.SemaphoreType.DMA,
            pltpu.SemaphoreType.DMA,
            pltpu.SemaphoreType.DMA,
            pltpu.SemaphoreType.DMA,
            pltpu.SemaphoreType.DMA,
        ],
        name="lightgcn_fused3",
    )


def kernel(edge_index, edge_weight, user_emb, item_emb):
    src = edge_index[1].astype(jnp.int32)
    dst = edge_index[0].astype(jnp.int32)
    src_p = src + PADG * (src >= N_U).astype(jnp.int32)
    dst_p = dst + PADG * (dst >= N_U).astype(jnp.int32)
    wbits = lax.bitcast_convert_type(edge_weight.astype(jnp.float32), jnp.int32)

    pad_idx = N_U + (jnp.arange(E_PAD - E, dtype=jnp.int32) % PADG)

    def pack(src_c):
        p = jnp.stack([src_c, dst_p, wbits])
        pads = jnp.stack([pad_idx, pad_idx,
                          jnp.zeros((E_PAD - E,), jnp.int32)])
        p = jnp.concatenate([p, pads], axis=1)
        return p.reshape(3, NBALL, B).transpose(1, 0, 2)

    packed = jnp.concatenate([pack(src_p), pack(src_p + NTOT)], axis=0)

    zpad = jnp.zeros((PADG, EMB), jnp.float32)
    emb0 = jnp.concatenate([user_emb, zpad, item_emb, zpad], axis=0)
    split0 = jnp.concatenate([emb0[:, :HEMB], emb0[:, HEMB:]], axis=0)

    _, _, sum3 = _make_fused()(packed, split0)

    out = jnp.concatenate([sum3[:NTOT], sum3[NTOT:]], axis=1)
    return (out[:N_U], out[HALF:HALF + N_I])
